# baseline (speedup 1.0000x reference)
import functools

import jax
import jax.numpy as jnp
from jax import lax
from jax.experimental import pallas as pl
from jax.experimental.pallas import tpu as pltpu
from jax.experimental.pallas import tpu_sc as plsc

N = 10000
E = 320000
D = 128
HD = D // 2
G = 64

NC = 2
NS = 16
NW = NC * NS
CH = 64
EPW = 10240
NCH = EPW // CH
EPAD = NW * EPW - E
NP = 10240
RPT = NP // NS
ZR = 128
RB = 8
DEGW = 16

BR = 1000

_mesh = plsc.VectorSubcoreMesh(core_axis_name="c", subcore_axis_name="s")
_f32 = jnp.float32


def _fill(ref, rows, width, value):
    per_row = width // 16

    def body(i, _):
        r = i // per_row
        c = (i % per_row) * 16
        ref[r, pl.ds(c, 16)] = jnp.full((16,), value, _f32)
        return 0

    lax.fori_loop(0, rows * per_row, body, 0)


@functools.partial(
    pl.kernel,
    out_type=jax.ShapeDtypeStruct((NC, NP, DEGW), _f32),
    mesh=_mesh,
    compiler_params=pltpu.CompilerParams(use_tc_tiling_on_sc=False),
    scratch_types=[
        pltpu.VMEM((NCH, CH), jnp.int32),
        pltpu.VMEM((CH, DEGW), _f32),
        pltpu.VMEM((ZR, DEGW), _f32),
        pltpu.VMEM_SHARED((NP, DEGW), _f32),
    ],
)
def _sc_degree(dst_hbm, out_hbm, idx_v, ones_v, zero_v, acc_s):
    c = lax.axis_index("c")
    s = lax.axis_index("s")
    wid = c * NS + s
    pltpu.sync_copy(dst_hbm.at[wid], idx_v)
    _fill(ones_v, CH, DEGW, 1.0)
    _fill(zero_v, ZR, DEGW, 0.0)
    for k in range(RPT // ZR):
        pltpu.sync_copy(zero_v, acc_s.at[pl.ds(s * RPT + k * ZR, ZR)])
    plsc.subcore_barrier()

    def body(j, _):
        pltpu.sync_copy(ones_v, acc_s.at[idx_v.at[j]], add=True)
        return 0

    lax.fori_loop(0, NCH, body, 0)
    plsc.subcore_barrier()
    pltpu.sync_copy(acc_s.at[pl.ds(s * RPT, RPT)],
                    out_hbm.at[c, pl.ds(s * RPT, RPT)])


@functools.partial(
    pl.kernel,
    out_type=[jax.ShapeDtypeStruct((NC, NP, HD), _f32),
              jax.ShapeDtypeStruct((NC, NP, HD), _f32)],
    mesh=_mesh,
    compiler_params=pltpu.CompilerParams(use_tc_tiling_on_sc=False),
    scratch_types=[
        pltpu.VMEM((NCH, CH), jnp.int32),
        pltpu.VMEM((NCH, CH), jnp.int32),
        pltpu.VMEM((RB, CH, HD), _f32),
        pltpu.VMEM((ZR, HD), _f32),
        pltpu.VMEM_SHARED((NP, HD), _f32),
        [pltpu.SemaphoreType.DMA] * RB,
        [pltpu.SemaphoreType.DMA] * RB,
    ],
)
def _sc_aggregate(glo_hbm, ghi_hbm, src_hbm, dst_hbm, outlo_hbm, outhi_hbm,
                  src_v, dst_v, rows_v, zero_v, acc_s, semg, semsc):
    c = lax.axis_index("c")
    s = lax.axis_index("s")
    wid = c * NS + s
    pltpu.sync_copy(src_hbm.at[wid], src_v)
    pltpu.sync_copy(dst_hbm.at[wid], dst_v)
    _fill(zero_v, ZR, HD, 0.0)

    for g_hbm, out_hbm in ((glo_hbm, outlo_hbm), (ghi_hbm, outhi_hbm)):
        for k in range(RPT // ZR):
            pltpu.sync_copy(zero_v, acc_s.at[pl.ds(s * RPT + k * ZR, ZR)])
        plsc.subcore_barrier()

        for b in range(RB // 2):
            pltpu.async_copy(g_hbm.at[src_v.at[b]], rows_v.at[b], semg[b])

        def body(t, _):
            for b in range(RB):
                j = t * RB + b
                b2 = (b + RB // 2) % RB
                pltpu.make_async_copy(g_hbm.at[src_v.at[j]],
                                      rows_v.at[b], semg[b]).wait()
                pltpu.async_copy(rows_v.at[b], acc_s.at[dst_v.at[j]],
                                 semsc[b], add=True)

                @pl.when(j >= RB // 2)
                def _():
                    pltpu.make_async_copy(rows_v.at[b2],
                                          acc_s.at[dst_v.at[j - RB // 2]],
                                          semsc[b2]).wait()

                @pl.when(j + RB // 2 < NCH)
                def _():
                    pltpu.async_copy(g_hbm.at[src_v.at[j + RB // 2]],
                                     rows_v.at[b2], semg[b2])
            return 0

        lax.fori_loop(0, NCH // RB, body, 0)
        for k in range(RB // 2):
            j = NCH - RB // 2 + k
            pltpu.make_async_copy(rows_v.at[j % RB], acc_s.at[dst_v.at[j]],
                                  semsc[j % RB]).wait()
        plsc.subcore_barrier()
        pltpu.sync_copy(acc_s.at[pl.ds(s * RPT, RPT)],
                        out_hbm.at[c, pl.ds(s * RPT, RPT)])
        plsc.subcore_barrier()


def _layer0_body(x_ref, w_ref, deg_ref, glo_ref, ghi_ref, dinv_ref):
    deg = deg_ref[0, :, 0:1] + deg_ref[1, :, 0:1] + 1.0
    dinv = lax.rsqrt(deg)
    dinv_ref[...] = dinv
    hw = jnp.dot(x_ref[...], w_ref[...], preferred_element_type=_f32) * dinv
    glo_ref[...] = hw[:, :HD]
    ghi_ref[...] = hw[:, HD:]


_layer0 = pl.pallas_call(
    _layer0_body,
    grid=(N // BR,),
    in_specs=[
        pl.BlockSpec((BR, D), lambda i: (i, 0)),
        pl.BlockSpec((D, D), lambda i: (0, 0)),
        pl.BlockSpec((NC, BR, DEGW), lambda i: (0, i, 0)),
    ],
    out_specs=[
        pl.BlockSpec((BR, HD), lambda i: (i, 0)),
        pl.BlockSpec((BR, HD), lambda i: (i, 0)),
        pl.BlockSpec((BR, 1), lambda i: (i, 0)),
    ],
    out_shape=[
        jax.ShapeDtypeStruct((N, HD), _f32),
        jax.ShapeDtypeStruct((N, HD), _f32),
        jax.ShapeDtypeStruct((N, 1), _f32),
    ],
)


def _combine(alo_ref, ahi_ref, glo_ref, ghi_ref, dinv_ref, b_ref):
    hl = alo_ref[0] + alo_ref[1] + glo_ref[...]
    hh = ahi_ref[0] + ahi_ref[1] + ghi_ref[...]
    h = jnp.concatenate([hl, hh], axis=1) * dinv_ref[...] + b_ref[...]
    return jnp.maximum(h, 0.0)


def _mid_body(alo_ref, ahi_ref, glo_ref, ghi_ref, dinv_ref, b_ref, w_ref,
              outlo_ref, outhi_ref):
    h = _combine(alo_ref, ahi_ref, glo_ref, ghi_ref, dinv_ref, b_ref)
    hw = jnp.dot(h, w_ref[...], preferred_element_type=_f32) * dinv_ref[...]
    outlo_ref[...] = hw[:, :HD]
    outhi_ref[...] = hw[:, HD:]


_mid = pl.pallas_call(
    _mid_body,
    grid=(N // BR,),
    in_specs=[
        pl.BlockSpec((NC, BR, HD), lambda i: (0, i, 0)),
        pl.BlockSpec((NC, BR, HD), lambda i: (0, i, 0)),
        pl.BlockSpec((BR, HD), lambda i: (i, 0)),
        pl.BlockSpec((BR, HD), lambda i: (i, 0)),
        pl.BlockSpec((BR, 1), lambda i: (i, 0)),
        pl.BlockSpec((1, D), lambda i: (0, 0)),
        pl.BlockSpec((D, D), lambda i: (0, 0)),
    ],
    out_specs=[
        pl.BlockSpec((BR, HD), lambda i: (i, 0)),
        pl.BlockSpec((BR, HD), lambda i: (i, 0)),
    ],
    out_shape=[
        jax.ShapeDtypeStruct((N, HD), _f32),
        jax.ShapeDtypeStruct((N, HD), _f32),
    ],
)


def _tail_body(alo_ref, ahi_ref, glo_ref, ghi_ref, dinv_ref, b_ref, batch_ref,
               wm1_ref, bm1_ref, wm2_ref, bm2_ref, wm3_ref, bm3_ref,
               out_ref, pooled_ref):
    i = pl.program_id(0)

    @pl.when(i == 0)
    def _():
        pooled_ref[...] = jnp.zeros_like(pooled_ref)

    h = _combine(alo_ref, ahi_ref, glo_ref, ghi_ref, dinv_ref, b_ref)
    seg = (batch_ref[...] ==
           lax.broadcasted_iota(jnp.int32, (BR, G), 1)).astype(_f32)
    pooled_ref[...] += lax.dot_general(seg, h, (((0,), (0,)), ((), ())),
                                       preferred_element_type=_f32)

    @pl.when(i == pl.num_programs(0) - 1)
    def _():
        z = jnp.maximum(jnp.dot(pooled_ref[...], wm1_ref[...],
                                preferred_element_type=_f32) + bm1_ref[...],
                        0.0)
        z = jnp.maximum(jnp.dot(z, wm2_ref[...],
                                preferred_element_type=_f32) + bm2_ref[...],
                        0.0)
        logits = jnp.dot(z, wm3_ref[...],
                         preferred_element_type=_f32) + bm3_ref[...]
        out_ref[...] = jax.nn.sigmoid(logits)


_tail = pl.pallas_call(
    _tail_body,
    grid=(N // BR,),
    in_specs=[
        pl.BlockSpec((NC, BR, HD), lambda i: (0, i, 0)),
        pl.BlockSpec((NC, BR, HD), lambda i: (0, i, 0)),
        pl.BlockSpec((BR, HD), lambda i: (i, 0)),
        pl.BlockSpec((BR, HD), lambda i: (i, 0)),
        pl.BlockSpec((BR, 1), lambda i: (i, 0)),
        pl.BlockSpec((1, D), lambda i: (0, 0)),
        pl.BlockSpec((BR, 1), lambda i: (i, 0)),
        pl.BlockSpec((D, D), lambda i: (0, 0)),
        pl.BlockSpec((1, D), lambda i: (0, 0)),
        pl.BlockSpec((D, D), lambda i: (0, 0)),
        pl.BlockSpec((1, D), lambda i: (0, 0)),
        pl.BlockSpec((D, 1), lambda i: (0, 0)),
        pl.BlockSpec((1, 1), lambda i: (0, 0)),
    ],
    out_specs=pl.BlockSpec((G, 1), lambda i: (0, 0)),
    out_shape=jax.ShapeDtypeStruct((G, 1), _f32),
    scratch_shapes=[pltpu.VMEM((G, D), _f32)],
)


def kernel(x, edge_index, batch, W0, b0, W1, b1, W2, b2,
           Wm1, bm1, Wm2, bm2, Wm3, bm3):
    src2 = jnp.concatenate(
        [edge_index[0],
         jnp.arange(EPAD, dtype=jnp.int32) * 977 % N]).reshape(NW, NCH, CH)
    dst2 = jnp.concatenate(
        [edge_index[1],
         N + jnp.arange(EPAD, dtype=jnp.int32) % (NP - N)]).reshape(
             NW, NCH, CH)
    batch2 = batch.reshape(N, 1)

    deg = _sc_degree(dst2)
    glo, ghi, dinv = _layer0(x, W0, deg)
    alo, ahi = _sc_aggregate(glo, ghi, src2, dst2)
    glo, ghi = _mid(alo, ahi, glo, ghi, dinv, b0.reshape(1, D), W1)
    alo, ahi = _sc_aggregate(glo, ghi, src2, dst2)
    glo, ghi = _mid(alo, ahi, glo, ghi, dinv, b1.reshape(1, D), W2)
    alo, ahi = _sc_aggregate(glo, ghi, src2, dst2)
    return _tail(alo, ahi, glo, ghi, dinv, b2.reshape(1, D), batch2,
                 Wm1, bm1.reshape(1, D), Wm2, bm2.reshape(1, D),
                 Wm3, bm3.reshape(1, 1))


# speedup vs baseline: 1.0251x; 1.0251x over previous
import functools

import jax
import jax.numpy as jnp
from jax import lax
from jax.experimental import pallas as pl
from jax.experimental.pallas import tpu as pltpu
from jax.experimental.pallas import tpu_sc as plsc

N = 10000
E = 320000
D = 128
HD = D // 2
G = 64

NC = 2
NS = 16
NW = NC * NS
CH = 80
EPW = 10240
NCH = EPW // CH
EPAD = NW * EPW - E
NP = 10240
RPT = NP // NS
ZR = 128
RB = 8
DEGW = 16

BR = 1000

_mesh = plsc.VectorSubcoreMesh(core_axis_name="c", subcore_axis_name="s")
_f32 = jnp.float32


def _fill(ref, rows, width, value):
    per_row = width // 16

    def body(i, _):
        r = i // per_row
        c = (i % per_row) * 16
        ref[r, pl.ds(c, 16)] = jnp.full((16,), value, _f32)
        return 0

    lax.fori_loop(0, rows * per_row, body, 0)


@functools.partial(
    pl.kernel,
    out_type=jax.ShapeDtypeStruct((NC, NP, DEGW), _f32),
    mesh=_mesh,
    compiler_params=pltpu.CompilerParams(use_tc_tiling_on_sc=False),
    scratch_types=[
        pltpu.VMEM((NCH, CH), jnp.int32),
        pltpu.VMEM((CH, DEGW), _f32),
        pltpu.VMEM((ZR, DEGW), _f32),
        pltpu.VMEM_SHARED((NP, DEGW), _f32),
    ],
)
def _sc_degree(dst_hbm, out_hbm, idx_v, ones_v, zero_v, acc_s):
    c = lax.axis_index("c")
    s = lax.axis_index("s")
    wid = c * NS + s
    pltpu.sync_copy(dst_hbm.at[wid], idx_v)
    _fill(ones_v, CH, DEGW, 1.0)
    _fill(zero_v, ZR, DEGW, 0.0)
    for k in range(RPT // ZR):
        pltpu.sync_copy(zero_v, acc_s.at[pl.ds(s * RPT + k * ZR, ZR)])
    plsc.subcore_barrier()

    def body(j, _):
        pltpu.sync_copy(ones_v, acc_s.at[idx_v.at[j]], add=True)
        return 0

    lax.fori_loop(0, NCH, body, 0)
    plsc.subcore_barrier()
    pltpu.sync_copy(acc_s.at[pl.ds(s * RPT, RPT)],
                    out_hbm.at[c, pl.ds(s * RPT, RPT)])


@functools.partial(
    pl.kernel,
    out_type=[jax.ShapeDtypeStruct((NC, NP, HD), _f32),
              jax.ShapeDtypeStruct((NC, NP, HD), _f32)],
    mesh=_mesh,
    compiler_params=pltpu.CompilerParams(use_tc_tiling_on_sc=False),
    scratch_types=[
        pltpu.VMEM((NCH, CH), jnp.int32),
        pltpu.VMEM((NCH, CH), jnp.int32),
        pltpu.VMEM((RB, CH, HD), _f32),
        pltpu.VMEM((ZR, HD), _f32),
        pltpu.VMEM_SHARED((NP, HD), _f32),
        [pltpu.SemaphoreType.DMA] * RB,
        [pltpu.SemaphoreType.DMA] * RB,
    ],
)
def _sc_aggregate(glo_hbm, ghi_hbm, src_hbm, dst_hbm, outlo_hbm, outhi_hbm,
                  src_v, dst_v, rows_v, zero_v, acc_s, semg, semsc):
    c = lax.axis_index("c")
    s = lax.axis_index("s")
    wid = c * NS + s
    pltpu.sync_copy(src_hbm.at[wid], src_v)
    pltpu.sync_copy(dst_hbm.at[wid], dst_v)
    _fill(zero_v, ZR, HD, 0.0)

    for g_hbm, out_hbm in ((glo_hbm, outlo_hbm), (ghi_hbm, outhi_hbm)):
        for k in range(RPT // ZR):
            pltpu.sync_copy(zero_v, acc_s.at[pl.ds(s * RPT + k * ZR, ZR)])
        plsc.subcore_barrier()

        for b in range(RB // 2):
            pltpu.async_copy(g_hbm.at[src_v.at[b]], rows_v.at[b], semg[b])

        def body(t, _):
            for b in range(RB):
                j = t * RB + b
                b2 = (b + RB // 2) % RB
                pltpu.make_async_copy(g_hbm.at[src_v.at[j]],
                                      rows_v.at[b], semg[b]).wait()
                pltpu.async_copy(rows_v.at[b], acc_s.at[dst_v.at[j]],
                                 semsc[b], add=True)

                @pl.when(j >= RB // 2)
                def _():
                    pltpu.make_async_copy(rows_v.at[b2],
                                          acc_s.at[dst_v.at[j - RB // 2]],
                                          semsc[b2]).wait()

                @pl.when(j + RB // 2 < NCH)
                def _():
                    pltpu.async_copy(g_hbm.at[src_v.at[j + RB // 2]],
                                     rows_v.at[b2], semg[b2])
            return 0

        lax.fori_loop(0, NCH // RB, body, 0)
        for k in range(RB // 2):
            j = NCH - RB // 2 + k
            pltpu.make_async_copy(rows_v.at[j % RB], acc_s.at[dst_v.at[j]],
                                  semsc[j % RB]).wait()
        plsc.subcore_barrier()
        pltpu.sync_copy(acc_s.at[pl.ds(s * RPT, RPT)],
                        out_hbm.at[c, pl.ds(s * RPT, RPT)])
        plsc.subcore_barrier()


def _layer0_body(x_ref, w_ref, deg_ref, glo_ref, ghi_ref, dinv_ref):
    deg = deg_ref[0, :, 0:1] + deg_ref[1, :, 0:1] + 1.0
    dinv = lax.rsqrt(deg)
    dinv_ref[...] = dinv
    hw = jnp.dot(x_ref[...], w_ref[...], preferred_element_type=_f32) * dinv
    glo_ref[...] = hw[:, :HD]
    ghi_ref[...] = hw[:, HD:]


_layer0 = pl.pallas_call(
    _layer0_body,
    grid=(N // BR,),
    in_specs=[
        pl.BlockSpec((BR, D), lambda i: (i, 0)),
        pl.BlockSpec((D, D), lambda i: (0, 0)),
        pl.BlockSpec((NC, BR, DEGW), lambda i: (0, i, 0)),
    ],
    out_specs=[
        pl.BlockSpec((BR, HD), lambda i: (i, 0)),
        pl.BlockSpec((BR, HD), lambda i: (i, 0)),
        pl.BlockSpec((BR, 1), lambda i: (i, 0)),
    ],
    out_shape=[
        jax.ShapeDtypeStruct((N, HD), _f32),
        jax.ShapeDtypeStruct((N, HD), _f32),
        jax.ShapeDtypeStruct((N, 1), _f32),
    ],
)


def _combine(alo_ref, ahi_ref, glo_ref, ghi_ref, dinv_ref, b_ref):
    hl = alo_ref[0] + alo_ref[1] + glo_ref[...]
    hh = ahi_ref[0] + ahi_ref[1] + ghi_ref[...]
    h = jnp.concatenate([hl, hh], axis=1) * dinv_ref[...] + b_ref[...]
    return jnp.maximum(h, 0.0)


def _mid_body(alo_ref, ahi_ref, glo_ref, ghi_ref, dinv_ref, b_ref, w_ref,
              outlo_ref, outhi_ref):
    h = _combine(alo_ref, ahi_ref, glo_ref, ghi_ref, dinv_ref, b_ref)
    hw = jnp.dot(h, w_ref[...], preferred_element_type=_f32) * dinv_ref[...]
    outlo_ref[...] = hw[:, :HD]
    outhi_ref[...] = hw[:, HD:]


_mid = pl.pallas_call(
    _mid_body,
    grid=(N // BR,),
    in_specs=[
        pl.BlockSpec((NC, BR, HD), lambda i: (0, i, 0)),
        pl.BlockSpec((NC, BR, HD), lambda i: (0, i, 0)),
        pl.BlockSpec((BR, HD), lambda i: (i, 0)),
        pl.BlockSpec((BR, HD), lambda i: (i, 0)),
        pl.BlockSpec((BR, 1), lambda i: (i, 0)),
        pl.BlockSpec((1, D), lambda i: (0, 0)),
        pl.BlockSpec((D, D), lambda i: (0, 0)),
    ],
    out_specs=[
        pl.BlockSpec((BR, HD), lambda i: (i, 0)),
        pl.BlockSpec((BR, HD), lambda i: (i, 0)),
    ],
    out_shape=[
        jax.ShapeDtypeStruct((N, HD), _f32),
        jax.ShapeDtypeStruct((N, HD), _f32),
    ],
)


def _tail_body(alo_ref, ahi_ref, glo_ref, ghi_ref, dinv_ref, b_ref, batch_ref,
               wm1_ref, bm1_ref, wm2_ref, bm2_ref, wm3_ref, bm3_ref,
               out_ref, pooled_ref):
    i = pl.program_id(0)

    @pl.when(i == 0)
    def _():
        pooled_ref[...] = jnp.zeros_like(pooled_ref)

    h = _combine(alo_ref, ahi_ref, glo_ref, ghi_ref, dinv_ref, b_ref)
    seg = (batch_ref[...] ==
           lax.broadcasted_iota(jnp.int32, (BR, G), 1)).astype(_f32)
    pooled_ref[...] += lax.dot_general(seg, h, (((0,), (0,)), ((), ())),
                                       preferred_element_type=_f32)

    @pl.when(i == pl.num_programs(0) - 1)
    def _():
        z = jnp.maximum(jnp.dot(pooled_ref[...], wm1_ref[...],
                                preferred_element_type=_f32) + bm1_ref[...],
                        0.0)
        z = jnp.maximum(jnp.dot(z, wm2_ref[...],
                                preferred_element_type=_f32) + bm2_ref[...],
                        0.0)
        logits = jnp.dot(z, wm3_ref[...],
                         preferred_element_type=_f32) + bm3_ref[...]
        out_ref[...] = jax.nn.sigmoid(logits)


_tail = pl.pallas_call(
    _tail_body,
    grid=(N // BR,),
    in_specs=[
        pl.BlockSpec((NC, BR, HD), lambda i: (0, i, 0)),
        pl.BlockSpec((NC, BR, HD), lambda i: (0, i, 0)),
        pl.BlockSpec((BR, HD), lambda i: (i, 0)),
        pl.BlockSpec((BR, HD), lambda i: (i, 0)),
        pl.BlockSpec((BR, 1), lambda i: (i, 0)),
        pl.BlockSpec((1, D), lambda i: (0, 0)),
        pl.BlockSpec((BR, 1), lambda i: (i, 0)),
        pl.BlockSpec((D, D), lambda i: (0, 0)),
        pl.BlockSpec((1, D), lambda i: (0, 0)),
        pl.BlockSpec((D, D), lambda i: (0, 0)),
        pl.BlockSpec((1, D), lambda i: (0, 0)),
        pl.BlockSpec((D, 1), lambda i: (0, 0)),
        pl.BlockSpec((1, 1), lambda i: (0, 0)),
    ],
    out_specs=pl.BlockSpec((G, 1), lambda i: (0, 0)),
    out_shape=jax.ShapeDtypeStruct((G, 1), _f32),
    scratch_shapes=[pltpu.VMEM((G, D), _f32)],
)


def kernel(x, edge_index, batch, W0, b0, W1, b1, W2, b2,
           Wm1, bm1, Wm2, bm2, Wm3, bm3):
    src2 = jnp.concatenate(
        [edge_index[0],
         jnp.arange(EPAD, dtype=jnp.int32) * 977 % N]).reshape(NW, NCH, CH)
    dst2 = jnp.concatenate(
        [edge_index[1],
         N + jnp.arange(EPAD, dtype=jnp.int32) % (NP - N)]).reshape(
             NW, NCH, CH)
    batch2 = batch.reshape(N, 1)

    deg = _sc_degree(dst2)
    glo, ghi, dinv = _layer0(x, W0, deg)
    alo, ahi = _sc_aggregate(glo, ghi, src2, dst2)
    glo, ghi = _mid(alo, ahi, glo, ghi, dinv, b0.reshape(1, D), W1)
    alo, ahi = _sc_aggregate(glo, ghi, src2, dst2)
    glo, ghi = _mid(alo, ahi, glo, ghi, dinv, b1.reshape(1, D), W2)
    alo, ahi = _sc_aggregate(glo, ghi, src2, dst2)
    return _tail(alo, ahi, glo, ghi, dinv, b2.reshape(1, D), batch2,
                 Wm1, bm1.reshape(1, D), Wm2, bm2.reshape(1, D),
                 Wm3, bm3.reshape(1, 1))


# speedup vs baseline: 1.0339x; 1.0086x over previous
import functools

import jax
import jax.numpy as jnp
from jax import lax
from jax.experimental import pallas as pl
from jax.experimental.pallas import tpu as pltpu
from jax.experimental.pallas import tpu_sc as plsc

N = 10000
E = 320000
D = 128
HD = D // 2
G = 64

NC = 2
NS = 16
NW = NC * NS
CH = 80
EPW = 10240
NCH = EPW // CH
EPAD = NW * EPW - E
NP = 10240
RPT = NP // NS
ZR = 128
RB = 8
DEGW = 16

BR = 1000

_mesh = plsc.VectorSubcoreMesh(core_axis_name="c", subcore_axis_name="s")
_f32 = jnp.float32


def _fill(ref, rows, width, value):
    per_row = width // 16

    def body(i, _):
        r = i // per_row
        c = (i % per_row) * 16
        ref[r, pl.ds(c, 16)] = jnp.full((16,), value, _f32)
        return 0

    lax.fori_loop(0, rows * per_row, body, 0)


@functools.partial(
    pl.kernel,
    out_type=jax.ShapeDtypeStruct((NC, NP, DEGW), _f32),
    mesh=_mesh,
    compiler_params=pltpu.CompilerParams(use_tc_tiling_on_sc=False),
    scratch_types=[
        pltpu.VMEM((NCH, CH), jnp.int32),
        pltpu.VMEM((CH, DEGW), _f32),
        pltpu.VMEM((ZR, DEGW), _f32),
        pltpu.VMEM_SHARED((NP, DEGW), _f32),
    ],
)
def _sc_degree(dst_hbm, out_hbm, idx_v, ones_v, zero_v, acc_s):
    c = lax.axis_index("c")
    s = lax.axis_index("s")
    wid = c * NS + s
    pltpu.sync_copy(dst_hbm.at[wid], idx_v)
    _fill(ones_v, CH, DEGW, 1.0)
    _fill(zero_v, ZR, DEGW, 0.0)
    for k in range(RPT // ZR):
        pltpu.sync_copy(zero_v, acc_s.at[pl.ds(s * RPT + k * ZR, ZR)])
    plsc.subcore_barrier()

    def body(j, _):
        pltpu.sync_copy(ones_v, acc_s.at[idx_v.at[j]], add=True)
        return 0

    lax.fori_loop(0, NCH, body, 0)
    plsc.subcore_barrier()
    pltpu.sync_copy(acc_s.at[pl.ds(s * RPT, RPT)],
                    out_hbm.at[c, pl.ds(s * RPT, RPT)])


@functools.partial(
    pl.kernel,
    out_type=[jax.ShapeDtypeStruct((NC, NP, HD), _f32),
              jax.ShapeDtypeStruct((NC, NP, HD), _f32)],
    mesh=_mesh,
    compiler_params=pltpu.CompilerParams(use_tc_tiling_on_sc=False),
    scratch_types=[
        pltpu.VMEM((NCH, CH), jnp.int32),
        pltpu.VMEM((NCH, CH), jnp.int32),
        pltpu.VMEM((RB, CH, HD), _f32),
        pltpu.VMEM((ZR, HD), _f32),
        pltpu.VMEM_SHARED((NP, HD), _f32),
        [pltpu.SemaphoreType.DMA] * RB,
        [pltpu.SemaphoreType.DMA] * RB,
    ],
)
def _sc_aggregate(glo_hbm, ghi_hbm, src_hbm, dst_hbm, outlo_hbm, outhi_hbm,
                  src_v, dst_v, rows_v, zero_v, acc_s, semg, semsc):
    c = lax.axis_index("c")
    s = lax.axis_index("s")
    wid = c * NS + s
    pltpu.sync_copy(src_hbm.at[wid], src_v)
    pltpu.sync_copy(dst_hbm.at[wid], dst_v)
    _fill(zero_v, ZR, HD, 0.0)

    def prime(g_hbm):
        for b in range(RB // 2):
            pltpu.async_copy(g_hbm.at[src_v.at[b]], rows_v.at[b], semg[b])

    def run_pass(g_hbm):
        def body(t, _):
            for b in range(RB):
                j = t * RB + b
                b2 = (b + RB // 2) % RB
                pltpu.make_async_copy(g_hbm.at[src_v.at[j]],
                                      rows_v.at[b], semg[b]).wait()
                pltpu.async_copy(rows_v.at[b], acc_s.at[dst_v.at[j]],
                                 semsc[b], add=True)

                @pl.when(j >= RB // 2)
                def _():
                    pltpu.make_async_copy(rows_v.at[b2],
                                          acc_s.at[dst_v.at[j - RB // 2]],
                                          semsc[b2]).wait()

                @pl.when(j + RB // 2 < NCH)
                def _():
                    pltpu.async_copy(g_hbm.at[src_v.at[j + RB // 2]],
                                     rows_v.at[b2], semg[b2])
            return 0

        lax.fori_loop(0, NCH // RB, body, 0)
        for k in range(RB // 2):
            j = NCH - RB // 2 + k
            pltpu.make_async_copy(rows_v.at[j % RB], acc_s.at[dst_v.at[j]],
                                  semsc[j % RB]).wait()
        plsc.subcore_barrier()

    def zero_acc():
        for k in range(RPT // ZR):
            pltpu.sync_copy(zero_v, acc_s.at[pl.ds(s * RPT + k * ZR, ZR)])

    def drain(out_hbm):
        pltpu.sync_copy(acc_s.at[pl.ds(s * RPT, RPT)],
                        out_hbm.at[c, pl.ds(s * RPT, RPT)])

    zero_acc()
    plsc.subcore_barrier()
    prime(glo_hbm)
    run_pass(glo_hbm)
    prime(ghi_hbm)
    drain(outlo_hbm)
    zero_acc()
    plsc.subcore_barrier()
    run_pass(ghi_hbm)
    drain(outhi_hbm)


def _layer0_body(x_ref, w_ref, deg_ref, glo_ref, ghi_ref, dinv_ref):
    deg = deg_ref[0, :, 0:1] + deg_ref[1, :, 0:1] + 1.0
    dinv = lax.rsqrt(deg)
    dinv_ref[...] = dinv
    hw = jnp.dot(x_ref[...], w_ref[...], preferred_element_type=_f32) * dinv
    glo_ref[...] = hw[:, :HD]
    ghi_ref[...] = hw[:, HD:]


_layer0 = pl.pallas_call(
    _layer0_body,
    grid=(N // BR,),
    in_specs=[
        pl.BlockSpec((BR, D), lambda i: (i, 0)),
        pl.BlockSpec((D, D), lambda i: (0, 0)),
        pl.BlockSpec((NC, BR, DEGW), lambda i: (0, i, 0)),
    ],
    out_specs=[
        pl.BlockSpec((BR, HD), lambda i: (i, 0)),
        pl.BlockSpec((BR, HD), lambda i: (i, 0)),
        pl.BlockSpec((BR, 1), lambda i: (i, 0)),
    ],
    out_shape=[
        jax.ShapeDtypeStruct((N, HD), _f32),
        jax.ShapeDtypeStruct((N, HD), _f32),
        jax.ShapeDtypeStruct((N, 1), _f32),
    ],
)


def _combine(alo_ref, ahi_ref, glo_ref, ghi_ref, dinv_ref, b_ref):
    hl = alo_ref[0] + alo_ref[1] + glo_ref[...]
    hh = ahi_ref[0] + ahi_ref[1] + ghi_ref[...]
    h = jnp.concatenate([hl, hh], axis=1) * dinv_ref[...] + b_ref[...]
    return jnp.maximum(h, 0.0)


def _mid_body(alo_ref, ahi_ref, glo_ref, ghi_ref, dinv_ref, b_ref, w_ref,
              outlo_ref, outhi_ref):
    h = _combine(alo_ref, ahi_ref, glo_ref, ghi_ref, dinv_ref, b_ref)
    hw = jnp.dot(h, w_ref[...], preferred_element_type=_f32) * dinv_ref[...]
    outlo_ref[...] = hw[:, :HD]
    outhi_ref[...] = hw[:, HD:]


_mid = pl.pallas_call(
    _mid_body,
    grid=(N // BR,),
    in_specs=[
        pl.BlockSpec((NC, BR, HD), lambda i: (0, i, 0)),
        pl.BlockSpec((NC, BR, HD), lambda i: (0, i, 0)),
        pl.BlockSpec((BR, HD), lambda i: (i, 0)),
        pl.BlockSpec((BR, HD), lambda i: (i, 0)),
        pl.BlockSpec((BR, 1), lambda i: (i, 0)),
        pl.BlockSpec((1, D), lambda i: (0, 0)),
        pl.BlockSpec((D, D), lambda i: (0, 0)),
    ],
    out_specs=[
        pl.BlockSpec((BR, HD), lambda i: (i, 0)),
        pl.BlockSpec((BR, HD), lambda i: (i, 0)),
    ],
    out_shape=[
        jax.ShapeDtypeStruct((N, HD), _f32),
        jax.ShapeDtypeStruct((N, HD), _f32),
    ],
)


def _tail_body(alo_ref, ahi_ref, glo_ref, ghi_ref, dinv_ref, b_ref, batch_ref,
               wm1_ref, bm1_ref, wm2_ref, bm2_ref, wm3_ref, bm3_ref,
               out_ref, pooled_ref):
    i = pl.program_id(0)

    @pl.when(i == 0)
    def _():
        pooled_ref[...] = jnp.zeros_like(pooled_ref)

    h = _combine(alo_ref, ahi_ref, glo_ref, ghi_ref, dinv_ref, b_ref)
    seg = (batch_ref[...] ==
           lax.broadcasted_iota(jnp.int32, (BR, G), 1)).astype(_f32)
    pooled_ref[...] += lax.dot_general(seg, h, (((0,), (0,)), ((), ())),
                                       preferred_element_type=_f32)

    @pl.when(i == pl.num_programs(0) - 1)
    def _():
        z = jnp.maximum(jnp.dot(pooled_ref[...], wm1_ref[...],
                                preferred_element_type=_f32) + bm1_ref[...],
                        0.0)
        z = jnp.maximum(jnp.dot(z, wm2_ref[...],
                                preferred_element_type=_f32) + bm2_ref[...],
                        0.0)
        logits = jnp.dot(z, wm3_ref[...],
                         preferred_element_type=_f32) + bm3_ref[...]
        out_ref[...] = jax.nn.sigmoid(logits)


_tail = pl.pallas_call(
    _tail_body,
    grid=(N // BR,),
    in_specs=[
        pl.BlockSpec((NC, BR, HD), lambda i: (0, i, 0)),
        pl.BlockSpec((NC, BR, HD), lambda i: (0, i, 0)),
        pl.BlockSpec((BR, HD), lambda i: (i, 0)),
        pl.BlockSpec((BR, HD), lambda i: (i, 0)),
        pl.BlockSpec((BR, 1), lambda i: (i, 0)),
        pl.BlockSpec((1, D), lambda i: (0, 0)),
        pl.BlockSpec((BR, 1), lambda i: (i, 0)),
        pl.BlockSpec((D, D), lambda i: (0, 0)),
        pl.BlockSpec((1, D), lambda i: (0, 0)),
        pl.BlockSpec((D, D), lambda i: (0, 0)),
        pl.BlockSpec((1, D), lambda i: (0, 0)),
        pl.BlockSpec((D, 1), lambda i: (0, 0)),
        pl.BlockSpec((1, 1), lambda i: (0, 0)),
    ],
    out_specs=pl.BlockSpec((G, 1), lambda i: (0, 0)),
    out_shape=jax.ShapeDtypeStruct((G, 1), _f32),
    scratch_shapes=[pltpu.VMEM((G, D), _f32)],
)


def kernel(x, edge_index, batch, W0, b0, W1, b1, W2, b2,
           Wm1, bm1, Wm2, bm2, Wm3, bm3):
    src2 = jnp.concatenate(
        [edge_index[0],
         jnp.arange(EPAD, dtype=jnp.int32) * 977 % N]).reshape(NW, NCH, CH)
    dst2 = jnp.concatenate(
        [edge_index[1],
         N + jnp.arange(EPAD, dtype=jnp.int32) % (NP - N)]).reshape(
             NW, NCH, CH)
    batch2 = batch.reshape(N, 1)

    deg = _sc_degree(dst2)
    glo, ghi, dinv = _layer0(x, W0, deg)
    alo, ahi = _sc_aggregate(glo, ghi, src2, dst2)
    glo, ghi = _mid(alo, ahi, glo, ghi, dinv, b0.reshape(1, D), W1)
    alo, ahi = _sc_aggregate(glo, ghi, src2, dst2)
    glo, ghi = _mid(alo, ahi, glo, ghi, dinv, b1.reshape(1, D), W2)
    alo, ahi = _sc_aggregate(glo, ghi, src2, dst2)
    return _tail(alo, ahi, glo, ghi, dinv, b2.reshape(1, D), batch2,
                 Wm1, bm1.reshape(1, D), Wm2, bm2.reshape(1, D),
                 Wm3, bm3.reshape(1, 1))


# speedup vs baseline: 1.0466x; 1.0122x over previous
import functools

import jax
import jax.numpy as jnp
from jax import lax
from jax.experimental import pallas as pl
from jax.experimental.pallas import tpu as pltpu
from jax.experimental.pallas import tpu_sc as plsc

N = 10000
E = 320000
D = 128
HD = D // 2
G = 64

NC = 2
NS = 16
NW = NC * NS
CH = 80
EPW = 10240
NCH = EPW // CH
EPAD = NW * EPW - E
NP = 10240
RPT = NP // NS
ZR = 128
RB = 8
DEGW = 16

BR = 1000

_mesh = plsc.VectorSubcoreMesh(core_axis_name="c", subcore_axis_name="s")
_f32 = jnp.float32


def _fill(ref, rows, width, value):
    per_row = width // 16

    def body(i, _):
        r = i // per_row
        c = (i % per_row) * 16
        ref[r, pl.ds(c, 16)] = jnp.full((16,), value, _f32)
        return 0

    lax.fori_loop(0, rows * per_row, body, 0)


@functools.partial(
    pl.kernel,
    out_type=jax.ShapeDtypeStruct((NC, NP, DEGW), _f32),
    mesh=_mesh,
    compiler_params=pltpu.CompilerParams(use_tc_tiling_on_sc=False),
    scratch_types=[
        pltpu.VMEM((NCH, CH), jnp.int32),
        pltpu.VMEM((CH, DEGW), _f32),
        pltpu.VMEM((ZR, DEGW), _f32),
        pltpu.VMEM_SHARED((NP, DEGW), _f32),
        [pltpu.SemaphoreType.DMA] * RB,
    ],
)
def _sc_degree(dst_hbm, out_hbm, idx_v, ones_v, zero_v, acc_s, semd):
    c = lax.axis_index("c")
    s = lax.axis_index("s")
    wid = c * NS + s
    pltpu.sync_copy(dst_hbm.at[wid], idx_v)
    _fill(ones_v, CH, DEGW, 1.0)
    _fill(zero_v, ZR, DEGW, 0.0)
    for k in range(RPT // ZR):
        pltpu.sync_copy(zero_v, acc_s.at[pl.ds(s * RPT + k * ZR, ZR)])
    plsc.subcore_barrier()

    def body(t, _):
        for b in range(RB):
            j = t * RB + b
            pltpu.async_copy(ones_v, acc_s.at[idx_v.at[j]], semd[b],
                             add=True)

            @pl.when(j >= RB)
            def _():
                pltpu.make_async_copy(ones_v, acc_s.at[idx_v.at[j - RB]],
                                      semd[b]).wait()
        return 0

    lax.fori_loop(0, NCH // RB, body, 0)
    for k in range(RB):
        j = NCH - RB + k
        pltpu.make_async_copy(ones_v, acc_s.at[idx_v.at[j]],
                              semd[j % RB]).wait()
    plsc.subcore_barrier()
    pltpu.sync_copy(acc_s.at[pl.ds(s * RPT, RPT)],
                    out_hbm.at[c, pl.ds(s * RPT, RPT)])


@functools.partial(
    pl.kernel,
    out_type=[jax.ShapeDtypeStruct((NC, NP, HD), _f32),
              jax.ShapeDtypeStruct((NC, NP, HD), _f32)],
    mesh=_mesh,
    compiler_params=pltpu.CompilerParams(use_tc_tiling_on_sc=False),
    scratch_types=[
        pltpu.VMEM((NCH, CH), jnp.int32),
        pltpu.VMEM((NCH, CH), jnp.int32),
        pltpu.VMEM((RB, CH, HD), _f32),
        pltpu.VMEM((ZR, HD), _f32),
        pltpu.VMEM_SHARED((NP, HD), _f32),
        [pltpu.SemaphoreType.DMA] * RB,
        [pltpu.SemaphoreType.DMA] * RB,
    ],
)
def _sc_aggregate(glo_hbm, ghi_hbm, src_hbm, dst_hbm, outlo_hbm, outhi_hbm,
                  src_v, dst_v, rows_v, zero_v, acc_s, semg, semsc):
    c = lax.axis_index("c")
    s = lax.axis_index("s")
    wid = c * NS + s
    pltpu.sync_copy(src_hbm.at[wid], src_v)
    pltpu.sync_copy(dst_hbm.at[wid], dst_v)
    _fill(zero_v, ZR, HD, 0.0)

    def prime(g_hbm):
        for b in range(RB // 2):
            pltpu.async_copy(g_hbm.at[src_v.at[b]], rows_v.at[b], semg[b])

    def run_pass(g_hbm):
        def body(t, _):
            for b in range(RB):
                j = t * RB + b
                b2 = (b + RB // 2) % RB
                pltpu.make_async_copy(g_hbm.at[src_v.at[j]],
                                      rows_v.at[b], semg[b]).wait()
                pltpu.async_copy(rows_v.at[b], acc_s.at[dst_v.at[j]],
                                 semsc[b], add=True)

                @pl.when(j >= RB // 2)
                def _():
                    pltpu.make_async_copy(rows_v.at[b2],
                                          acc_s.at[dst_v.at[j - RB // 2]],
                                          semsc[b2]).wait()

                @pl.when(j + RB // 2 < NCH)
                def _():
                    pltpu.async_copy(g_hbm.at[src_v.at[j + RB // 2]],
                                     rows_v.at[b2], semg[b2])
            return 0

        lax.fori_loop(0, NCH // RB, body, 0)
        for k in range(RB // 2):
            j = NCH - RB // 2 + k
            pltpu.make_async_copy(rows_v.at[j % RB], acc_s.at[dst_v.at[j]],
                                  semsc[j % RB]).wait()
        plsc.subcore_barrier()

    def zero_acc():
        for k in range(RPT // ZR):
            pltpu.sync_copy(zero_v, acc_s.at[pl.ds(s * RPT + k * ZR, ZR)])

    def drain(out_hbm):
        pltpu.sync_copy(acc_s.at[pl.ds(s * RPT, RPT)],
                        out_hbm.at[c, pl.ds(s * RPT, RPT)])

    zero_acc()
    plsc.subcore_barrier()
    prime(glo_hbm)
    run_pass(glo_hbm)
    prime(ghi_hbm)
    drain(outlo_hbm)
    zero_acc()
    plsc.subcore_barrier()
    run_pass(ghi_hbm)
    drain(outhi_hbm)


def _layer0_body(x_ref, w_ref, deg_ref, glo_ref, ghi_ref, dinv_ref):
    deg = deg_ref[0, :, 0:1] + deg_ref[1, :, 0:1] + 1.0
    dinv = lax.rsqrt(deg)
    dinv_ref[...] = dinv
    hw = jnp.dot(x_ref[...], w_ref[...], preferred_element_type=_f32) * dinv
    glo_ref[...] = hw[:, :HD]
    ghi_ref[...] = hw[:, HD:]


_layer0 = pl.pallas_call(
    _layer0_body,
    grid=(N // BR,),
    in_specs=[
        pl.BlockSpec((BR, D), lambda i: (i, 0)),
        pl.BlockSpec((D, D), lambda i: (0, 0)),
        pl.BlockSpec((NC, BR, DEGW), lambda i: (0, i, 0)),
    ],
    out_specs=[
        pl.BlockSpec((BR, HD), lambda i: (i, 0)),
        pl.BlockSpec((BR, HD), lambda i: (i, 0)),
        pl.BlockSpec((BR, 1), lambda i: (i, 0)),
    ],
    out_shape=[
        jax.ShapeDtypeStruct((N, HD), _f32),
        jax.ShapeDtypeStruct((N, HD), _f32),
        jax.ShapeDtypeStruct((N, 1), _f32),
    ],
)


def _combine(alo_ref, ahi_ref, glo_ref, ghi_ref, dinv_ref, b_ref):
    hl = alo_ref[0] + alo_ref[1] + glo_ref[...]
    hh = ahi_ref[0] + ahi_ref[1] + ghi_ref[...]
    h = jnp.concatenate([hl, hh], axis=1) * dinv_ref[...] + b_ref[...]
    return jnp.maximum(h, 0.0)


def _mid_body(alo_ref, ahi_ref, glo_ref, ghi_ref, dinv_ref, b_ref, w_ref,
              outlo_ref, outhi_ref):
    h = _combine(alo_ref, ahi_ref, glo_ref, ghi_ref, dinv_ref, b_ref)
    hw = jnp.dot(h, w_ref[...], preferred_element_type=_f32) * dinv_ref[...]
    outlo_ref[...] = hw[:, :HD]
    outhi_ref[...] = hw[:, HD:]


_mid = pl.pallas_call(
    _mid_body,
    grid=(N // BR,),
    in_specs=[
        pl.BlockSpec((NC, BR, HD), lambda i: (0, i, 0)),
        pl.BlockSpec((NC, BR, HD), lambda i: (0, i, 0)),
        pl.BlockSpec((BR, HD), lambda i: (i, 0)),
        pl.BlockSpec((BR, HD), lambda i: (i, 0)),
        pl.BlockSpec((BR, 1), lambda i: (i, 0)),
        pl.BlockSpec((1, D), lambda i: (0, 0)),
        pl.BlockSpec((D, D), lambda i: (0, 0)),
    ],
    out_specs=[
        pl.BlockSpec((BR, HD), lambda i: (i, 0)),
        pl.BlockSpec((BR, HD), lambda i: (i, 0)),
    ],
    out_shape=[
        jax.ShapeDtypeStruct((N, HD), _f32),
        jax.ShapeDtypeStruct((N, HD), _f32),
    ],
)


def _tail_body(alo_ref, ahi_ref, glo_ref, ghi_ref, dinv_ref, b_ref, batch_ref,
               wm1_ref, bm1_ref, wm2_ref, bm2_ref, wm3_ref, bm3_ref,
               out_ref, pooled_ref):
    i = pl.program_id(0)

    @pl.when(i == 0)
    def _():
        pooled_ref[...] = jnp.zeros_like(pooled_ref)

    h = _combine(alo_ref, ahi_ref, glo_ref, ghi_ref, dinv_ref, b_ref)
    seg = (batch_ref[...] ==
           lax.broadcasted_iota(jnp.int32, (BR, G), 1)).astype(_f32)
    pooled_ref[...] += lax.dot_general(seg, h, (((0,), (0,)), ((), ())),
                                       preferred_element_type=_f32)

    @pl.when(i == pl.num_programs(0) - 1)
    def _():
        z = jnp.maximum(jnp.dot(pooled_ref[...], wm1_ref[...],
                                preferred_element_type=_f32) + bm1_ref[...],
                        0.0)
        z = jnp.maximum(jnp.dot(z, wm2_ref[...],
                                preferred_element_type=_f32) + bm2_ref[...],
                        0.0)
        logits = jnp.dot(z, wm3_ref[...],
                         preferred_element_type=_f32) + bm3_ref[...]
        out_ref[...] = jax.nn.sigmoid(logits)


_tail = pl.pallas_call(
    _tail_body,
    grid=(N // BR,),
    in_specs=[
        pl.BlockSpec((NC, BR, HD), lambda i: (0, i, 0)),
        pl.BlockSpec((NC, BR, HD), lambda i: (0, i, 0)),
        pl.BlockSpec((BR, HD), lambda i: (i, 0)),
        pl.BlockSpec((BR, HD), lambda i: (i, 0)),
        pl.BlockSpec((BR, 1), lambda i: (i, 0)),
        pl.BlockSpec((1, D), lambda i: (0, 0)),
        pl.BlockSpec((BR, 1), lambda i: (i, 0)),
        pl.BlockSpec((D, D), lambda i: (0, 0)),
        pl.BlockSpec((1, D), lambda i: (0, 0)),
        pl.BlockSpec((D, D), lambda i: (0, 0)),
        pl.BlockSpec((1, D), lambda i: (0, 0)),
        pl.BlockSpec((D, 1), lambda i: (0, 0)),
        pl.BlockSpec((1, 1), lambda i: (0, 0)),
    ],
    out_specs=pl.BlockSpec((G, 1), lambda i: (0, 0)),
    out_shape=jax.ShapeDtypeStruct((G, 1), _f32),
    scratch_shapes=[pltpu.VMEM((G, D), _f32)],
)


def kernel(x, edge_index, batch, W0, b0, W1, b1, W2, b2,
           Wm1, bm1, Wm2, bm2, Wm3, bm3):
    src2 = jnp.concatenate(
        [edge_index[0],
         jnp.arange(EPAD, dtype=jnp.int32) * 977 % N]).reshape(NW, NCH, CH)
    dst2 = jnp.concatenate(
        [edge_index[1],
         N + jnp.arange(EPAD, dtype=jnp.int32) % (NP - N)]).reshape(
             NW, NCH, CH)
    batch2 = batch.reshape(N, 1)

    deg = _sc_degree(dst2)
    glo, ghi, dinv = _layer0(x, W0, deg)
    alo, ahi = _sc_aggregate(glo, ghi, src2, dst2)
    glo, ghi = _mid(alo, ahi, glo, ghi, dinv, b0.reshape(1, D), W1)
    alo, ahi = _sc_aggregate(glo, ghi, src2, dst2)
    glo, ghi = _mid(alo, ahi, glo, ghi, dinv, b1.reshape(1, D), W2)
    alo, ahi = _sc_aggregate(glo, ghi, src2, dst2)
    return _tail(alo, ahi, glo, ghi, dinv, b2.reshape(1, D), batch2,
                 Wm1, bm1.reshape(1, D), Wm2, bm2.reshape(1, D),
                 Wm3, bm3.reshape(1, 1))
